# Initial kernel scaffold; baseline (speedup 1.0000x reference)
import jax
import jax.numpy as jnp
from jax.experimental import pallas as pl


def kernel(x, edge_index, W1, b1, gamma1, beta1, W2, b2, gamma2, beta2, W3, b3):
    raise NotImplementedError("write your pallas kernel here")


# speedup vs baseline: 10.9456x; 10.9456x over previous
import functools

import jax
import jax.numpy as jnp
from jax import lax
from jax.experimental import pallas as pl
from jax.experimental.pallas import tpu as pltpu
from jax.experimental.pallas import tpu_sc as plsc

_N = 10000
_E = 320000
_NPAD = 10240
_D = 128
_EPS = 1e-5
_CH = 80
_NW = 32
_EPW = _E // _NW
_RPS = _NPAD // 16
_BLK = 1280
_GRID = _NPAD // _BLK

_MESH = plsc.VectorSubcoreMesh(
    core_axis_name="c", subcore_axis_name="s", num_cores=2, num_subcores=16
)


@functools.partial(
    pl.kernel,
    out_type=jax.ShapeDtypeStruct((2, _NPAD, 16), jnp.float32),
    mesh=_MESH,
    scratch_types=[
        pltpu.VMEM((_CH,), jnp.int32),
        pltpu.VMEM((_CH, 16), jnp.float32),
        pltpu.VMEM((_CH, 16), jnp.float32),
        pltpu.VMEM_SHARED((_NPAD, 16), jnp.float32),
    ],
)
def _sc_degree(dst_hbm, out_hbm, idx_d, ones_v, zeros_v, acc):
    c = lax.axis_index("c")
    s = lax.axis_index("s")
    wid = c * 16 + s

    def init_row(i, _):
        ones_v[i, :] = jnp.full((16,), 1.0, jnp.float32)
        zeros_v[i, :] = jnp.zeros((16,), jnp.float32)
        return 0

    lax.fori_loop(0, _CH, init_row, 0)
    base_rows = s * _RPS

    def zacc(k, _):
        pltpu.sync_copy(zeros_v, acc.at[pl.ds(base_rows + k * _CH, _CH)])
        return 0

    lax.fori_loop(0, _RPS // _CH, zacc, 0)
    plsc.subcore_barrier()

    ebase = wid * _EPW

    def chunk(i, _):
        b = pl.multiple_of(ebase + i * _CH, 8)
        pltpu.sync_copy(dst_hbm.at[pl.ds(b, _CH)], idx_d)
        pltpu.sync_copy(ones_v, acc.at[idx_d], add=True)
        return 0

    lax.fori_loop(0, _EPW // _CH, chunk, 0)
    plsc.subcore_barrier()
    pltpu.sync_copy(
        acc.at[pl.ds(base_rows, _RPS)], out_hbm.at[c, pl.ds(base_rows, _RPS)]
    )


@functools.partial(
    pl.kernel,
    out_type=jax.ShapeDtypeStruct((2, _NPAD, _D), jnp.float32),
    mesh=_MESH,
    scratch_types=[
        pltpu.VMEM((_CH,), jnp.int32),
        pltpu.VMEM((_CH,), jnp.int32),
        pltpu.VMEM((_CH, _D), jnp.float32),
        pltpu.VMEM_SHARED((_NPAD, _D), jnp.float32),
        pltpu.SemaphoreType.DMA,
    ],
)
def _sc_scatter(p_hbm, src_hbm, dst_hbm, out_hbm, idx_s, idx_d, rows, acc, sem):
    c = lax.axis_index("c")
    s = lax.axis_index("s")
    wid = c * 16 + s

    def zero_row(i, _):
        for j in range(_D // 16):
            rows[i, pl.ds(j * 16, 16)] = jnp.zeros((16,), jnp.float32)
        return 0

    lax.fori_loop(0, _CH, zero_row, 0)
    base_rows = s * _RPS

    def zacc(k, _):
        pltpu.sync_copy(rows, acc.at[pl.ds(base_rows + k * _CH, _CH)])
        return 0

    lax.fori_loop(0, _RPS // _CH, zacc, 0)
    plsc.subcore_barrier()

    ebase = wid * _EPW

    def chunk(i, _):
        b = pl.multiple_of(ebase + i * _CH, 8)
        pltpu.sync_copy(src_hbm.at[pl.ds(b, _CH)], idx_s)
        pltpu.sync_copy(dst_hbm.at[pl.ds(b, _CH)], idx_d)
        pltpu.async_copy(p_hbm.at[idx_s], rows, sem).wait()
        pltpu.sync_copy(rows, acc.at[idx_d], add=True)
        return 0

    lax.fori_loop(0, _EPW // _CH, chunk, 0)
    plsc.subcore_barrier()
    pltpu.sync_copy(
        acc.at[pl.ds(base_rows, _RPS)], out_hbm.at[c, pl.ds(base_rows, _RPS)]
    )


def _tc_prep1(degp, xp, W1):
    def body(degp_ref, x_ref, w_ref, dinv_ref, p_ref):
        deg = degp_ref[0] + degp_ref[1]
        dinv = lax.rsqrt(deg[:, 0:1] + 1.0)
        dinv_ref[...] = dinv
        h = jnp.dot(x_ref[...], w_ref[...], preferred_element_type=jnp.float32)
        p_ref[...] = h * dinv

    return pl.pallas_call(
        body,
        grid=(_GRID,),
        in_specs=[
            pl.BlockSpec((2, _BLK, 16), lambda i: (0, i, 0)),
            pl.BlockSpec((_BLK, _D), lambda i: (i, 0)),
            pl.BlockSpec((_D, _D), lambda i: (0, 0)),
        ],
        out_specs=[
            pl.BlockSpec((_BLK, 1), lambda i: (i, 0)),
            pl.BlockSpec((_BLK, _D), lambda i: (i, 0)),
        ],
        out_shape=[
            jax.ShapeDtypeStruct((_NPAD, 1), jnp.float32),
            jax.ShapeDtypeStruct((_NPAD, _D), jnp.float32),
        ],
    )(degp, xp, W1)


def _tc_stats(S, P, dinv, b):
    def body(s_ref, p_ref, dinv_ref, b_ref, c_ref, st_ref):
        i = pl.program_id(0)
        C = dinv_ref[...] * (s_ref[0] + s_ref[1] + p_ref[...]) + b_ref[...]
        c_ref[...] = C
        rid = lax.broadcasted_iota(jnp.int32, (_BLK, _D), 0) + i * _BLK
        Cm = jnp.where(rid < _N, C, 0.0)

        @pl.when(i == 0)
        def _():
            st_ref[...] = jnp.zeros((8, _D), jnp.float32)

        st_ref[0:1, :] = st_ref[0:1, :] + jnp.sum(Cm, axis=0, keepdims=True)
        st_ref[1:2, :] = st_ref[1:2, :] + jnp.sum(Cm * Cm, axis=0, keepdims=True)

    return pl.pallas_call(
        body,
        grid=(_GRID,),
        in_specs=[
            pl.BlockSpec((2, _BLK, _D), lambda i: (0, i, 0)),
            pl.BlockSpec((_BLK, _D), lambda i: (i, 0)),
            pl.BlockSpec((_BLK, 1), lambda i: (i, 0)),
            pl.BlockSpec((1, _D), lambda i: (0, 0)),
        ],
        out_specs=[
            pl.BlockSpec((_BLK, _D), lambda i: (i, 0)),
            pl.BlockSpec((8, _D), lambda i: (0, 0)),
        ],
        out_shape=[
            jax.ShapeDtypeStruct((_NPAD, _D), jnp.float32),
            jax.ShapeDtypeStruct((8, _D), jnp.float32),
        ],
    )(S, P, dinv, b)


def _bn_coeffs(st_ref, g_ref, be_ref):
    mu = st_ref[0:1, :] * (1.0 / _N)
    var = st_ref[1:2, :] * (1.0 / _N) - mu * mu
    a = g_ref[...] * lax.rsqrt(var + _EPS)
    cc = be_ref[...] - mu * a
    return a, cc


def _tc_next(C, st, gamma, beta, dinv, Wn):
    def body(c_ref, st_ref, g_ref, be_ref, dinv_ref, w_ref, out_ref):
        a, cc = _bn_coeffs(st_ref, g_ref, be_ref)
        X = jnp.maximum(c_ref[...] * a + cc, 0.0)
        h = jnp.dot(X, w_ref[...], preferred_element_type=jnp.float32)
        out_ref[...] = h * dinv_ref[...]

    return pl.pallas_call(
        body,
        grid=(_GRID,),
        in_specs=[
            pl.BlockSpec((_BLK, _D), lambda i: (i, 0)),
            pl.BlockSpec((8, _D), lambda i: (0, 0)),
            pl.BlockSpec((1, _D), lambda i: (0, 0)),
            pl.BlockSpec((1, _D), lambda i: (0, 0)),
            pl.BlockSpec((_BLK, 1), lambda i: (i, 0)),
            pl.BlockSpec((_D, _D), lambda i: (0, 0)),
        ],
        out_specs=pl.BlockSpec((_BLK, _D), lambda i: (i, 0)),
        out_shape=jax.ShapeDtypeStruct((_NPAD, _D), jnp.float32),
    )(C, st, gamma, beta, dinv, Wn)


def _tc_next_nomat(C, st, gamma, beta, dinv):
    def body(c_ref, st_ref, g_ref, be_ref, dinv_ref, out_ref):
        a, cc = _bn_coeffs(st_ref, g_ref, be_ref)
        X = jnp.maximum(c_ref[...] * a + cc, 0.0)
        out_ref[...] = X * dinv_ref[...]

    return pl.pallas_call(
        body,
        grid=(_GRID,),
        in_specs=[
            pl.BlockSpec((_BLK, _D), lambda i: (i, 0)),
            pl.BlockSpec((8, _D), lambda i: (0, 0)),
            pl.BlockSpec((1, _D), lambda i: (0, 0)),
            pl.BlockSpec((1, _D), lambda i: (0, 0)),
            pl.BlockSpec((_BLK, 1), lambda i: (i, 0)),
        ],
        out_specs=pl.BlockSpec((_BLK, _D), lambda i: (i, 0)),
        out_shape=jax.ShapeDtypeStruct((_NPAD, _D), jnp.float32),
    )(C, st, gamma, beta, dinv)


def _tc_final(S, P, dinv, W3p, b3p):
    def body(s_ref, p_ref, dinv_ref, w_ref, b_ref, out_ref):
        C3 = dinv_ref[...] * (s_ref[0] + s_ref[1] + p_ref[...])
        logits = jnp.dot(C3, w_ref[...], preferred_element_type=jnp.float32)
        logits = logits + b_ref[...]
        m = jnp.max(logits, axis=1, keepdims=True)
        e = jnp.exp(logits - m)
        ssum = jnp.sum(e, axis=1, keepdims=True)
        out_ref[...] = logits - m - jnp.log(ssum)

    return pl.pallas_call(
        body,
        grid=(_GRID,),
        in_specs=[
            pl.BlockSpec((2, _BLK, _D), lambda i: (0, i, 0)),
            pl.BlockSpec((_BLK, _D), lambda i: (i, 0)),
            pl.BlockSpec((_BLK, 1), lambda i: (i, 0)),
            pl.BlockSpec((_D, _D), lambda i: (0, 0)),
            pl.BlockSpec((1, _D), lambda i: (0, 0)),
        ],
        out_specs=pl.BlockSpec((_BLK, _D), lambda i: (i, 0)),
        out_shape=jax.ShapeDtypeStruct((_NPAD, _D), jnp.float32),
    )(S, P, dinv, W3p, b3p)


def kernel(x, edge_index, W1, b1, gamma1, beta1, W2, b2, gamma2, beta2, W3, b3):
    src = edge_index[0]
    dst = edge_index[1]
    xp = jnp.pad(x, ((0, _NPAD - _N), (0, 0)))

    degp = _sc_degree(dst)
    dinv, P1 = _tc_prep1(degp, xp, W1)

    S1 = _sc_scatter(P1, src, dst)
    C1, st1 = _tc_stats(S1, P1, dinv, b1.reshape(1, -1))
    P2 = _tc_next(C1, st1, gamma1.reshape(1, -1), beta1.reshape(1, -1), dinv, W2)

    S2 = _sc_scatter(P2, src, dst)
    C2, st2 = _tc_stats(S2, P2, dinv, b2.reshape(1, -1))
    P3 = _tc_next_nomat(C2, st2, gamma2.reshape(1, -1), beta2.reshape(1, -1), dinv)

    S3 = _sc_scatter(P3, src, dst)
    W3p = jnp.pad(W3, ((0, 0), (0, _D - W3.shape[1])))
    b3p = jnp.concatenate(
        [b3, jnp.full((_D - b3.shape[0],), -1e30, jnp.float32)]
    ).reshape(1, -1)
    out = _tc_final(S3, P3, dinv, W3p, b3p)
    return out[:_N, : W3.shape[1]]


# speedup vs baseline: 24.8246x; 2.2680x over previous
import functools

import jax
import jax.numpy as jnp
from jax import lax
from jax.experimental import pallas as pl
from jax.experimental.pallas import tpu as pltpu
from jax.experimental.pallas import tpu_sc as plsc

_N = 10000
_E = 320000
_NPAD = 10240
_D = 128
_EPS = 1e-5
_CH = 100
_NCH = 100
_G = 20
_NW = 32
_EPW = _E // _NW
_RPS = _NPAD // 16
_BLK = 1280
_GRID = _NPAD // _BLK

_MESH = plsc.VectorSubcoreMesh(
    core_axis_name="c", subcore_axis_name="s", num_cores=2, num_subcores=16
)


@functools.partial(
    pl.kernel,
    out_type=jax.ShapeDtypeStruct((2, _NPAD, 16), jnp.float32),
    mesh=_MESH,
    scratch_types=[
        pltpu.VMEM((_NCH, _CH), jnp.int32),
        pltpu.VMEM((_CH, 16), jnp.float32),
        pltpu.VMEM((_CH, 16), jnp.float32),
        pltpu.VMEM_SHARED((_NPAD, 16), jnp.float32),
    ],
)
def _sc_degree(dst_hbm, out_hbm, idx_d, ones_v, zeros_v, acc):
    c = lax.axis_index("c")
    s = lax.axis_index("s")
    wid = c * 16 + s

    def init_row(i, _):
        ones_v[i, :] = jnp.full((16,), 1.0, jnp.float32)
        zeros_v[i, :] = jnp.zeros((16,), jnp.float32)
        return 0

    lax.fori_loop(0, _CH, init_row, 0)
    base_rows = s * _RPS

    def zacc(k, _):
        pltpu.sync_copy(zeros_v, acc.at[pl.ds(base_rows + k * _CH, _CH)])
        return 0

    lax.fori_loop(0, 6, zacc, 0)
    pltpu.sync_copy(zeros_v.at[pl.ds(0, 40)], acc.at[pl.ds(base_rows + 600, 40)])
    pltpu.sync_copy(dst_hbm.at[wid], idx_d)
    plsc.subcore_barrier()

    def chunk(i, _):
        pltpu.sync_copy(ones_v, acc.at[idx_d.at[i]], add=True)
        return 0

    lax.fori_loop(0, _NCH, chunk, 0)
    plsc.subcore_barrier()
    pltpu.sync_copy(
        acc.at[pl.ds(base_rows, _RPS)], out_hbm.at[c, pl.ds(base_rows, _RPS)]
    )


@functools.partial(
    pl.kernel,
    out_type=jax.ShapeDtypeStruct((2, _NPAD, _D), jnp.float32),
    mesh=_MESH,
    scratch_types=[
        pltpu.VMEM((_G, _CH), jnp.int32),
        pltpu.VMEM((_G, _CH), jnp.int32),
        pltpu.VMEM((2, _CH, _D), jnp.float32),
        pltpu.VMEM_SHARED((_NPAD, _D), jnp.float32),
        pltpu.SemaphoreType.DMA,
        pltpu.SemaphoreType.DMA,
    ],
)
def _sc_scatter(p_hbm, src_hbm, dst_hbm, out_hbm, idx_s, idx_d, rows, acc,
                sem0, sem1):
    c = lax.axis_index("c")
    s = lax.axis_index("s")
    wid = c * 16 + s

    def zero_row(i, _):
        for j in range(_D // 16):
            rows[0, i, pl.ds(j * 16, 16)] = jnp.zeros((16,), jnp.float32)
        return 0

    lax.fori_loop(0, _CH, zero_row, 0)
    base_rows = s * _RPS

    def zacc(k, _):
        pltpu.sync_copy(rows.at[0], acc.at[pl.ds(base_rows + k * _CH, _CH)])
        return 0

    lax.fori_loop(0, 6, zacc, 0)
    pltpu.sync_copy(rows.at[0, pl.ds(0, 40)], acc.at[pl.ds(base_rows + 600, 40)])
    plsc.subcore_barrier()

    sems = (sem0, sem1)

    def group(g, _):
        pltpu.sync_copy(src_hbm.at[wid, g], idx_s)
        pltpu.sync_copy(dst_hbm.at[wid, g], idx_d)
        pltpu.async_copy(p_hbm.at[idx_s.at[0]], rows.at[0], sem0)

        def pair(k, _):
            for b in range(2):
                i = k * 2 + b

                @pl.when(i + 1 < _G)
                def _():
                    pltpu.async_copy(
                        p_hbm.at[idx_s.at[i + 1]], rows.at[1 - b], sems[1 - b]
                    )

                pltpu.make_async_copy(
                    p_hbm.at[idx_s.at[i]], rows.at[b], sems[b]
                ).wait()
                pltpu.sync_copy(rows.at[b], acc.at[idx_d.at[i]], add=True)
            return 0

        lax.fori_loop(0, _G // 2, pair, 0)
        return 0

    lax.fori_loop(0, _NCH // _G, group, 0)
    plsc.subcore_barrier()
    pltpu.sync_copy(
        acc.at[pl.ds(base_rows, _RPS)], out_hbm.at[c, pl.ds(base_rows, _RPS)]
    )


def _tc_prep1(degp, xp, W1):
    def body(degp_ref, x_ref, w_ref, dinv_ref, p_ref):
        deg = degp_ref[0] + degp_ref[1]
        dinv = lax.rsqrt(deg[:, 0:1] + 1.0)
        dinv_ref[...] = dinv
        h = jnp.dot(x_ref[...], w_ref[...], preferred_element_type=jnp.float32)
        p_ref[...] = h * dinv

    return pl.pallas_call(
        body,
        grid=(_GRID,),
        in_specs=[
            pl.BlockSpec((2, _BLK, 16), lambda i: (0, i, 0)),
            pl.BlockSpec((_BLK, _D), lambda i: (i, 0)),
            pl.BlockSpec((_D, _D), lambda i: (0, 0)),
        ],
        out_specs=[
            pl.BlockSpec((_BLK, 1), lambda i: (i, 0)),
            pl.BlockSpec((_BLK, _D), lambda i: (i, 0)),
        ],
        out_shape=[
            jax.ShapeDtypeStruct((_NPAD, 1), jnp.float32),
            jax.ShapeDtypeStruct((_NPAD, _D), jnp.float32),
        ],
    )(degp, xp, W1)


def _tc_stats(S, P, dinv, b):
    def body(s_ref, p_ref, dinv_ref, b_ref, c_ref, st_ref):
        i = pl.program_id(0)
        C = dinv_ref[...] * (s_ref[0] + s_ref[1] + p_ref[...]) + b_ref[...]
        c_ref[...] = C
        rid = lax.broadcasted_iota(jnp.int32, (_BLK, _D), 0) + i * _BLK
        Cm = jnp.where(rid < _N, C, 0.0)

        @pl.when(i == 0)
        def _():
            st_ref[...] = jnp.zeros((8, _D), jnp.float32)

        st_ref[0:1, :] = st_ref[0:1, :] + jnp.sum(Cm, axis=0, keepdims=True)
        st_ref[1:2, :] = st_ref[1:2, :] + jnp.sum(Cm * Cm, axis=0, keepdims=True)

    return pl.pallas_call(
        body,
        grid=(_GRID,),
        in_specs=[
            pl.BlockSpec((2, _BLK, _D), lambda i: (0, i, 0)),
            pl.BlockSpec((_BLK, _D), lambda i: (i, 0)),
            pl.BlockSpec((_BLK, 1), lambda i: (i, 0)),
            pl.BlockSpec((1, _D), lambda i: (0, 0)),
        ],
        out_specs=[
            pl.BlockSpec((_BLK, _D), lambda i: (i, 0)),
            pl.BlockSpec((8, _D), lambda i: (0, 0)),
        ],
        out_shape=[
            jax.ShapeDtypeStruct((_NPAD, _D), jnp.float32),
            jax.ShapeDtypeStruct((8, _D), jnp.float32),
        ],
    )(S, P, dinv, b)


def _bn_coeffs(st_ref, g_ref, be_ref):
    mu = st_ref[0:1, :] * (1.0 / _N)
    var = st_ref[1:2, :] * (1.0 / _N) - mu * mu
    a = g_ref[...] * lax.rsqrt(var + _EPS)
    cc = be_ref[...] - mu * a
    return a, cc


def _tc_next(C, st, gamma, beta, dinv, Wn):
    def body(c_ref, st_ref, g_ref, be_ref, dinv_ref, w_ref, out_ref):
        a, cc = _bn_coeffs(st_ref, g_ref, be_ref)
        X = jnp.maximum(c_ref[...] * a + cc, 0.0)
        h = jnp.dot(X, w_ref[...], preferred_element_type=jnp.float32)
        out_ref[...] = h * dinv_ref[...]

    return pl.pallas_call(
        body,
        grid=(_GRID,),
        in_specs=[
            pl.BlockSpec((_BLK, _D), lambda i: (i, 0)),
            pl.BlockSpec((8, _D), lambda i: (0, 0)),
            pl.BlockSpec((1, _D), lambda i: (0, 0)),
            pl.BlockSpec((1, _D), lambda i: (0, 0)),
            pl.BlockSpec((_BLK, 1), lambda i: (i, 0)),
            pl.BlockSpec((_D, _D), lambda i: (0, 0)),
        ],
        out_specs=pl.BlockSpec((_BLK, _D), lambda i: (i, 0)),
        out_shape=jax.ShapeDtypeStruct((_NPAD, _D), jnp.float32),
    )(C, st, gamma, beta, dinv, Wn)


def _tc_next_nomat(C, st, gamma, beta, dinv):
    def body(c_ref, st_ref, g_ref, be_ref, dinv_ref, out_ref):
        a, cc = _bn_coeffs(st_ref, g_ref, be_ref)
        X = jnp.maximum(c_ref[...] * a + cc, 0.0)
        out_ref[...] = X * dinv_ref[...]

    return pl.pallas_call(
        body,
        grid=(_GRID,),
        in_specs=[
            pl.BlockSpec((_BLK, _D), lambda i: (i, 0)),
            pl.BlockSpec((8, _D), lambda i: (0, 0)),
            pl.BlockSpec((1, _D), lambda i: (0, 0)),
            pl.BlockSpec((1, _D), lambda i: (0, 0)),
            pl.BlockSpec((_BLK, 1), lambda i: (i, 0)),
        ],
        out_specs=pl.BlockSpec((_BLK, _D), lambda i: (i, 0)),
        out_shape=jax.ShapeDtypeStruct((_NPAD, _D), jnp.float32),
    )(C, st, gamma, beta, dinv)


def _tc_final(S, P, dinv, W3p, b3p):
    def body(s_ref, p_ref, dinv_ref, w_ref, b_ref, out_ref):
        C3 = dinv_ref[...] * (s_ref[0] + s_ref[1] + p_ref[...])
        logits = jnp.dot(C3, w_ref[...], preferred_element_type=jnp.float32)
        logits = logits + b_ref[...]
        m = jnp.max(logits, axis=1, keepdims=True)
        e = jnp.exp(logits - m)
        ssum = jnp.sum(e, axis=1, keepdims=True)
        out_ref[...] = logits - m - jnp.log(ssum)

    return pl.pallas_call(
        body,
        grid=(_GRID,),
        in_specs=[
            pl.BlockSpec((2, _BLK, _D), lambda i: (0, i, 0)),
            pl.BlockSpec((_BLK, _D), lambda i: (i, 0)),
            pl.BlockSpec((_BLK, 1), lambda i: (i, 0)),
            pl.BlockSpec((_D, _D), lambda i: (0, 0)),
            pl.BlockSpec((1, _D), lambda i: (0, 0)),
        ],
        out_specs=pl.BlockSpec((_BLK, _D), lambda i: (i, 0)),
        out_shape=jax.ShapeDtypeStruct((_NPAD, _D), jnp.float32),
    )(S, P, dinv, W3p, b3p)


def kernel(x, edge_index, W1, b1, gamma1, beta1, W2, b2, gamma2, beta2, W3, b3):
    src = edge_index[0].reshape(_NW, _NCH // _G, _G, _CH)
    dst = edge_index[1].reshape(_NW, _NCH // _G, _G, _CH)
    dst_deg = edge_index[1].reshape(_NW, _NCH, _CH)
    xp = jnp.pad(x, ((0, _NPAD - _N), (0, 0)))

    degp = _sc_degree(dst_deg)
    dinv, P1 = _tc_prep1(degp, xp, W1)

    S1 = _sc_scatter(P1, src, dst)
    C1, st1 = _tc_stats(S1, P1, dinv, b1.reshape(1, -1))
    P2 = _tc_next(C1, st1, gamma1.reshape(1, -1), beta1.reshape(1, -1), dinv, W2)

    S2 = _sc_scatter(P2, src, dst)
    C2, st2 = _tc_stats(S2, P2, dinv, b2.reshape(1, -1))
    P3 = _tc_next_nomat(C2, st2, gamma2.reshape(1, -1), beta2.reshape(1, -1), dinv)

    S3 = _sc_scatter(P3, src, dst)
    W3p = jnp.pad(W3, ((0, 0), (0, _D - W3.shape[1])))
    b3p = jnp.concatenate(
        [b3, jnp.full((_D - b3.shape[0],), -1e30, jnp.float32)]
    ).reshape(1, -1)
    out = _tc_final(S3, P3, dinv, W3p, b3p)
    return out[:_N, : W3.shape[1]]
